# Initial kernel scaffold; baseline (speedup 1.0000x reference)
import jax
import jax.numpy as jnp
from jax.experimental import pallas as pl


def kernel(x, edge_index, edge_attr, u, batch, W_projH, b_projH, W1, b1, ln_w, ln_b, W2, b2):
    raise NotImplementedError("write your pallas kernel here")


# speedup vs baseline: 5.3803x; 5.3803x over previous
import functools

import jax
import jax.numpy as jnp
from jax import lax
from jax.experimental import pallas as pl
from jax.experimental.pallas import tpu as pltpu
from jax.experimental.pallas import tpu_sc as plsc

N = 10000
E = 320000
HID = 128
HEADS = 4
NG = 16

NC = 2
NS = 16
NW = NC * NS
EPW = E // NW
CH = 128
NCH = 79
TAIL = EPW - (NCH - 1) * CH
NP = 10112
RPT = NP // NS
L = 16
CHUNKS = [(k * CH, CH) for k in range(RPT // CH)]
if RPT % CH:
    CHUNKS.append((RPT - RPT % CH, RPT % CH))


def _iota16():
    return lax.broadcasted_iota(jnp.int32, (L,), 0)


def _full16(v):
    return jnp.full((L,), v, dtype=jnp.int32)


BE = 4000


def _proj_body(ea_ref, w_ref, b_ref, v_ref):
    v_ref[...] = (
        jnp.dot(ea_ref[...], w_ref[...], preferred_element_type=jnp.float32)
        + b_ref[...]
    )


def _proj(edge_attr, W_projH, b_projH):
    return pl.pallas_call(
        _proj_body,
        grid=(E // BE,),
        in_specs=[
            pl.BlockSpec((BE, HID), lambda i: (i, 0)),
            pl.BlockSpec((HID, 2 * HEADS), lambda i: (0, 0)),
            pl.BlockSpec((1, 2 * HEADS), lambda i: (0, 0)),
        ],
        out_specs=pl.BlockSpec((BE, 2 * HEADS), lambda i: (i, 0)),
        out_shape=jax.ShapeDtypeStruct((E, 2 * HEADS), jnp.float32),
    )(edge_attr, W_projH, b_projH.reshape(1, 2 * HEADS))


def _mesh():
    return plsc.VectorSubcoreMesh(
        core_axis_name="c", subcore_axis_name="s",
        num_cores=NC, num_subcores=NS)


def _sc_m_body(ea, colt, m_part, acc_m, a_buf, col_buf):
    c = lax.axis_index("c")
    s = lax.axis_index("s")
    wid = c * NS + s
    zeros = jnp.zeros((L,), jnp.float32)

    def _za(r, _):
        for k in range(HID // L):
            a_buf[r, pl.ds(k * L, L)] = zeros
        return _
    lax.fori_loop(0, CH, _za, 0)

    for off, sz in CHUNKS:
        pltpu.sync_copy(a_buf.at[pl.ds(0, sz)],
                        acc_m.at[pl.ds(s * RPT + off, sz)])

    plsc.subcore_barrier()

    def _step(i, _):
        e0 = wid * EPW + jnp.minimum(i * CH, EPW - CH)
        rw = wid * NCH + i
        pltpu.sync_copy(colt.at[rw], col_buf)
        pltpu.sync_copy(ea.at[pl.ds(e0, CH)], a_buf)
        pltpu.sync_copy(a_buf, acc_m.at[col_buf], add=True)
        return _
    lax.fori_loop(0, NCH, _step, 0)

    plsc.subcore_barrier()

    for off, sz in CHUNKS:
        r0 = s * RPT + off
        pltpu.sync_copy(acc_m.at[pl.ds(r0, sz)], a_buf.at[pl.ds(0, sz)])
        pltpu.sync_copy(a_buf.at[pl.ds(0, sz)], m_part.at[c, pl.ds(r0, sz)])


def _sc_m(edge_attr, colt):
    f = pl.kernel(
        _sc_m_body,
        out_type=jax.ShapeDtypeStruct((NC, NP, HID), jnp.float32),
        mesh=_mesh(),
        scratch_types=[
            pltpu.VMEM_SHARED((NP, HID), jnp.float32),
            pltpu.VMEM((CH, HID), jnp.float32),
            pltpu.VMEM((CH,), jnp.int32),
        ],
        compiler_params=pltpu.CompilerParams(needs_layout_passes=False),
    )
    return f(edge_attr, colt)


def _sc_w_body(vv, rowt, colt, posx, posy, w_part,
               acc_w, posx_v, posy_v, v_buf, pay_buf, row_buf, col_buf):
    c = lax.axis_index("c")
    s = lax.axis_index("s")
    wid = c * NS + s
    zeros = jnp.zeros((L,), jnp.float32)

    def _zp(r, _):
        pay_buf[r, :] = zeros
        return _
    lax.fori_loop(0, CH, _zp, 0)

    for off, sz in CHUNKS:
        pltpu.sync_copy(pay_buf.at[pl.ds(0, sz)],
                        acc_w.at[pl.ds(s * RPT + off, sz)])

    pltpu.sync_copy(posx, posx_v)
    pltpu.sync_copy(posy, posy_v)

    unit = jnp.where(_iota16() == HEADS, 1.0, 0.0).astype(jnp.float32)

    def _zu(r, _):
        pay_buf[r, :] = unit
        return _
    lax.fori_loop(0, CH, _zu, 0)

    plsc.subcore_barrier()

    def _step(i, _):
        e0 = wid * EPW + jnp.minimum(i * CH, EPW - CH)
        rw = wid * NCH + i
        pltpu.sync_copy(rowt.at[rw], row_buf)
        pltpu.sync_copy(colt.at[rw], col_buf)
        pltpu.sync_copy(vv.at[pl.ds(e0, CH)], v_buf)
        for j in range(CH // L):
            e16 = _iota16() + j * L
            r16 = row_buf[pl.ds(j * L, L)]
            psx = plsc.load_gather(posx_v, [r16])
            psy = plsc.load_gather(posy_v, [r16])
            for h in range(HEADS):
                vx = plsc.load_gather(v_buf, [e16, _full16(2 * h)])
                vy = plsc.load_gather(v_buf, [e16, _full16(2 * h + 1)])
                c1 = vx * psy - vy * psx
                plsc.store_scatter(pay_buf, [e16, _full16(h)], c1)
        pltpu.sync_copy(pay_buf, acc_w.at[col_buf], add=True)
        return _
    lax.fori_loop(0, NCH, _step, 0)

    plsc.subcore_barrier()

    for off, sz in CHUNKS:
        r0 = s * RPT + off
        pltpu.sync_copy(acc_w.at[pl.ds(r0, sz)], pay_buf.at[pl.ds(0, sz)])
        pltpu.sync_copy(pay_buf.at[pl.ds(0, sz)], w_part.at[c, pl.ds(r0, sz)])


def _sc_w(V, rowt, colt, posx, posy):
    f = pl.kernel(
        _sc_w_body,
        out_type=jax.ShapeDtypeStruct((NC, NP, L), jnp.float32),
        mesh=_mesh(),
        scratch_types=[
            pltpu.VMEM_SHARED((NP, L), jnp.float32),
            pltpu.VMEM((N,), jnp.float32),
            pltpu.VMEM((N,), jnp.float32),
            pltpu.VMEM((CH, 2 * HEADS), jnp.float32),
            pltpu.VMEM((CH, L), jnp.float32),
            pltpu.VMEM((CH,), jnp.int32),
            pltpu.VMEM((CH,), jnp.int32),
        ],
        compiler_params=pltpu.CompilerParams(needs_layout_passes=False),
    )
    return f(V, rowt, colt, posx, posy)


BN = 1000


def _mlp_body(he_ref, mp_ref, wp_ref, px_ref, py_ref, bt_ref, u_ref,
              wpx_ref, wpy_ref, bxy_ref, w1a_ref, w1b_ref, w1c_ref, w1d_ref,
              b1_ref, lnw_ref, lnb_ref, w2_ref, b2_ref, out_ref):
    m = mp_ref[0] + mp_ref[1]
    w = wp_ref[0] + wp_ref[1]
    c1 = w[:, :HEADS]
    deg = w[:, HEADS:HEADS + 1]
    bx = bxy_ref[0:1, :]
    by = bxy_ref[1:2, :]
    svx = jnp.dot(m, wpx_ref[...], preferred_element_type=jnp.float32) + deg * bx
    svy = jnp.dot(m, wpy_ref[...], preferred_element_type=jnp.float32) + deg * by
    curl = c1 + px_ref[...] * svy - py_ref[...] * svx
    onehot = (bt_ref[...] == lax.broadcasted_iota(jnp.int32, (BN, NG), 1)
              ).astype(jnp.float32)
    uw = jnp.dot(u_ref[...], w1d_ref[...], preferred_element_type=jnp.float32)
    h = (jnp.dot(he_ref[...], w1a_ref[...], preferred_element_type=jnp.float32)
         + jnp.dot(m, w1b_ref[...], preferred_element_type=jnp.float32)
         + jnp.dot(curl, w1c_ref[...], preferred_element_type=jnp.float32)
         + jnp.dot(onehot, uw, preferred_element_type=jnp.float32)
         + b1_ref[...])
    mean = jnp.mean(h, axis=-1, keepdims=True)
    var = jnp.mean((h - mean) ** 2, axis=-1, keepdims=True)
    h = (h - mean) * jax.lax.rsqrt(var + 1e-5) * lnw_ref[...] + lnb_ref[...]
    h = h * jax.nn.sigmoid(h)
    out_ref[...] = (
        jnp.dot(h, w2_ref[...], preferred_element_type=jnp.float32) + b2_ref[...])


def _mlp(h_E, m_part, w_part, px, py, batch2, u, Wpx, Wpy, bxy,
         W1a, W1b, W1c, W1d, b1, ln_w, ln_b, W2, b2):
    full = lambda shape: pl.BlockSpec(shape, lambda i: tuple(0 for _ in shape))
    return pl.pallas_call(
        _mlp_body,
        grid=(N // BN,),
        in_specs=[
            pl.BlockSpec((BN, HID), lambda i: (i, 0)),
            pl.BlockSpec((NC, BN, HID), lambda i: (0, i, 0)),
            pl.BlockSpec((NC, BN, L), lambda i: (0, i, 0)),
            pl.BlockSpec((BN, 1), lambda i: (i, 0)),
            pl.BlockSpec((BN, 1), lambda i: (i, 0)),
            pl.BlockSpec((BN, 1), lambda i: (i, 0)),
            full((NG, HID)),
            full((HID, HEADS)),
            full((HID, HEADS)),
            full((2, HEADS)),
            full((HID, HID)),
            full((HID, HID)),
            full((HEADS, HID)),
            full((HID, HID)),
            full((1, HID)),
            full((1, HID)),
            full((1, HID)),
            full((HID, HID)),
            full((1, HID)),
        ],
        out_specs=pl.BlockSpec((BN, HID), lambda i: (i, 0)),
        out_shape=jax.ShapeDtypeStruct((N, HID), jnp.float32),
    )(h_E, m_part, w_part, px, py, batch2, u, Wpx, Wpy, bxy,
      W1a, W1b, W1c, W1d, b1, ln_w, ln_b, W2, b2)


def kernel(x, edge_index, edge_attr, u, batch, W_projH, b_projH, W1, b1,
           ln_w, ln_b, W2, b2):
    h_E = x[:, :HID]
    posx = x[:, HID]
    posy = x[:, HID + 1]

    V = _proj(edge_attr, W_projH, b_projH)

    i_ids = jnp.arange(NCH)[None, :, None]
    j_ids = jnp.arange(CH)[None, None, :]
    idx3 = (jnp.arange(NW)[:, None, None] * EPW
            + jnp.minimum(i_ids * CH, EPW - CH) + j_ids)
    rowt = edge_index[0][idx3].reshape(NW * NCH, CH)
    valid = (i_ids < NCH - 1) | (j_ids >= CH - TAIL)
    colt = jnp.where(valid, edge_index[1][idx3], N).reshape(NW * NCH, CH)
    m_part = _sc_m(edge_attr, colt)
    w_part = _sc_w(V, rowt, colt, posx, posy)

    Wpx = W_projH[:, 0::2]
    Wpy = W_projH[:, 1::2]
    bxy = jnp.stack([b_projH[0::2], b_projH[1::2]])
    W1a = W1[:HID]
    W1b = W1[HID:2 * HID]
    W1c = W1[2 * HID:2 * HID + HEADS]
    W1d = W1[2 * HID + HEADS:]
    return _mlp(h_E, m_part, w_part, posx[:, None], posy[:, None],
                batch[:, None].astype(jnp.int32), u, Wpx, Wpy, bxy,
                W1a, W1b, W1c, W1d, b1.reshape(1, HID), ln_w.reshape(1, HID),
                ln_b.reshape(1, HID), W2, b2.reshape(1, HID))


# speedup vs baseline: 5.7188x; 1.0629x over previous
import functools

import jax
import jax.numpy as jnp
from jax import lax
from jax.experimental import pallas as pl
from jax.experimental.pallas import tpu as pltpu
from jax.experimental.pallas import tpu_sc as plsc

N = 10000
E = 320000
HID = 128
HEADS = 4
NG = 16

NC = 2
NS = 16
NW = NC * NS
CH = 128
NB = E // CH
NBI = -(-NB // NW)
NP = 10112
RPT = NP // NS
L = 16
CHUNKS = [(k * CH, CH) for k in range(RPT // CH)]
if RPT % CH:
    CHUNKS.append((RPT - RPT % CH, RPT % CH))


def _iota16():
    return lax.broadcasted_iota(jnp.int32, (L,), 0)


def _full16(v):
    return jnp.full((L,), v, dtype=jnp.int32)


BE = 4000


def _proj_body(ea_ref, w_ref, b_ref, v_ref):
    v_ref[...] = (
        jnp.dot(ea_ref[...], w_ref[...], preferred_element_type=jnp.float32)
        + b_ref[...]
    )


def _proj(edge_attr, W_projH, b_projH):
    return pl.pallas_call(
        _proj_body,
        grid=(E // BE,),
        in_specs=[
            pl.BlockSpec((BE, HID), lambda i: (i, 0)),
            pl.BlockSpec((HID, 2 * HEADS), lambda i: (0, 0)),
            pl.BlockSpec((1, 2 * HEADS), lambda i: (0, 0)),
        ],
        out_specs=pl.BlockSpec((BE, 2 * HEADS), lambda i: (i, 0)),
        out_shape=jax.ShapeDtypeStruct((E, 2 * HEADS), jnp.float32),
    )(edge_attr, W_projH, b_projH.reshape(1, 2 * HEADS))


def _mesh():
    return plsc.VectorSubcoreMesh(
        core_axis_name="c", subcore_axis_name="s",
        num_cores=NC, num_subcores=NS)


def _sc_m_body(ea, colt, m_part, acc_m, a_buf, col_buf):
    c = lax.axis_index("c")
    s = lax.axis_index("s")
    wid = c * NS + s
    zeros = jnp.zeros((L,), jnp.float32)

    def _za(r, _):
        for k in range(HID // L):
            a_buf[r, pl.ds(k * L, L)] = zeros
        return _
    lax.fori_loop(0, CH, _za, 0)

    for off, sz in CHUNKS:
        pltpu.sync_copy(a_buf.at[pl.ds(0, sz)],
                        acc_m.at[pl.ds(s * RPT + off, sz)])

    plsc.subcore_barrier()

    def _step(i, carry):
        rw = i * NW + wid

        @pl.when(rw < NB)
        def _body():
            pltpu.sync_copy(colt.at[rw], col_buf)
            pltpu.sync_copy(ea.at[pl.ds(rw * CH, CH)], a_buf)
            pltpu.sync_copy(a_buf, acc_m.at[col_buf], add=True)
        return carry
    lax.fori_loop(0, NBI, _step, 0)

    plsc.subcore_barrier()

    for off, sz in CHUNKS:
        r0 = s * RPT + off
        pltpu.sync_copy(acc_m.at[pl.ds(r0, sz)], a_buf.at[pl.ds(0, sz)])
        pltpu.sync_copy(a_buf.at[pl.ds(0, sz)], m_part.at[c, pl.ds(r0, sz)])


def _sc_m(edge_attr, colt):
    f = pl.kernel(
        _sc_m_body,
        out_type=jax.ShapeDtypeStruct((NC, NP, HID), jnp.float32),
        mesh=_mesh(),
        scratch_types=[
            pltpu.VMEM_SHARED((NP, HID), jnp.float32),
            pltpu.VMEM((CH, HID), jnp.float32),
            pltpu.VMEM((CH,), jnp.int32),
        ],
        compiler_params=pltpu.CompilerParams(needs_layout_passes=False),
    )
    return f(edge_attr, colt)


def _sc_w_body(vv, rowt, colt, posx, posy, w_part,
               acc_w, posx_v, posy_v, v_buf, pay_buf, row_buf, col_buf):
    c = lax.axis_index("c")
    s = lax.axis_index("s")
    wid = c * NS + s
    zeros = jnp.zeros((L,), jnp.float32)

    def _zp(r, _):
        pay_buf[r, :] = zeros
        return _
    lax.fori_loop(0, CH, _zp, 0)

    for off, sz in CHUNKS:
        pltpu.sync_copy(pay_buf.at[pl.ds(0, sz)],
                        acc_w.at[pl.ds(s * RPT + off, sz)])

    pltpu.sync_copy(posx, posx_v)
    pltpu.sync_copy(posy, posy_v)

    unit = jnp.where(_iota16() == HEADS, 1.0, 0.0).astype(jnp.float32)

    def _zu(r, _):
        pay_buf[r, :] = unit
        return _
    lax.fori_loop(0, CH, _zu, 0)

    plsc.subcore_barrier()

    def _step(i, carry):
        rw = i * NW + wid

        @pl.when(rw < NB)
        def _body():
            pltpu.sync_copy(rowt.at[rw], row_buf)
            pltpu.sync_copy(colt.at[rw], col_buf)
            pltpu.sync_copy(vv.at[pl.ds(rw * CH, CH)], v_buf)
            for j in range(CH // L):
                e16 = _iota16() + j * L
                r16 = row_buf[pl.ds(j * L, L)]
                psx = plsc.load_gather(posx_v, [r16])
                psy = plsc.load_gather(posy_v, [r16])
                for h in range(HEADS):
                    vx = plsc.load_gather(v_buf, [e16, _full16(2 * h)])
                    vy = plsc.load_gather(v_buf, [e16, _full16(2 * h + 1)])
                    c1 = vx * psy - vy * psx
                    plsc.store_scatter(pay_buf, [e16, _full16(h)], c1)
            pltpu.sync_copy(pay_buf, acc_w.at[col_buf], add=True)
        return carry
    lax.fori_loop(0, NBI, _step, 0)

    plsc.subcore_barrier()

    for off, sz in CHUNKS:
        r0 = s * RPT + off
        pltpu.sync_copy(acc_w.at[pl.ds(r0, sz)], pay_buf.at[pl.ds(0, sz)])
        pltpu.sync_copy(pay_buf.at[pl.ds(0, sz)], w_part.at[c, pl.ds(r0, sz)])


def _sc_w(V, rowt, colt, posx, posy):
    f = pl.kernel(
        _sc_w_body,
        out_type=jax.ShapeDtypeStruct((NC, NP, L), jnp.float32),
        mesh=_mesh(),
        scratch_types=[
            pltpu.VMEM_SHARED((NP, L), jnp.float32),
            pltpu.VMEM((N,), jnp.float32),
            pltpu.VMEM((N,), jnp.float32),
            pltpu.VMEM((CH, 2 * HEADS), jnp.float32),
            pltpu.VMEM((CH, L), jnp.float32),
            pltpu.VMEM((CH,), jnp.int32),
            pltpu.VMEM((CH,), jnp.int32),
        ],
        compiler_params=pltpu.CompilerParams(needs_layout_passes=False),
    )
    return f(V, rowt, colt, posx, posy)


BN = 1000


def _mlp_body(he_ref, mp_ref, wp_ref, px_ref, py_ref, bt_ref, u_ref,
              wpx_ref, wpy_ref, bxy_ref, w1a_ref, w1b_ref, w1c_ref, w1d_ref,
              b1_ref, lnw_ref, lnb_ref, w2_ref, b2_ref, out_ref):
    m = mp_ref[0] + mp_ref[1]
    w = wp_ref[0] + wp_ref[1]
    c1 = w[:, :HEADS]
    deg = w[:, HEADS:HEADS + 1]
    bx = bxy_ref[0:1, :]
    by = bxy_ref[1:2, :]
    svx = jnp.dot(m, wpx_ref[...], preferred_element_type=jnp.float32) + deg * bx
    svy = jnp.dot(m, wpy_ref[...], preferred_element_type=jnp.float32) + deg * by
    curl = c1 + px_ref[...] * svy - py_ref[...] * svx
    onehot = (bt_ref[...] == lax.broadcasted_iota(jnp.int32, (BN, NG), 1)
              ).astype(jnp.float32)
    uw = jnp.dot(u_ref[...], w1d_ref[...], preferred_element_type=jnp.float32)
    h = (jnp.dot(he_ref[...], w1a_ref[...], preferred_element_type=jnp.float32)
         + jnp.dot(m, w1b_ref[...], preferred_element_type=jnp.float32)
         + jnp.dot(curl, w1c_ref[...], preferred_element_type=jnp.float32)
         + jnp.dot(onehot, uw, preferred_element_type=jnp.float32)
         + b1_ref[...])
    mean = jnp.mean(h, axis=-1, keepdims=True)
    var = jnp.mean((h - mean) ** 2, axis=-1, keepdims=True)
    h = (h - mean) * jax.lax.rsqrt(var + 1e-5) * lnw_ref[...] + lnb_ref[...]
    h = h * jax.nn.sigmoid(h)
    out_ref[...] = (
        jnp.dot(h, w2_ref[...], preferred_element_type=jnp.float32) + b2_ref[...])


def _mlp(h_E, m_part, w_part, px, py, batch2, u, Wpx, Wpy, bxy,
         W1a, W1b, W1c, W1d, b1, ln_w, ln_b, W2, b2):
    full = lambda shape: pl.BlockSpec(shape, lambda i: tuple(0 for _ in shape))
    return pl.pallas_call(
        _mlp_body,
        grid=(N // BN,),
        in_specs=[
            pl.BlockSpec((BN, HID), lambda i: (i, 0)),
            pl.BlockSpec((NC, BN, HID), lambda i: (0, i, 0)),
            pl.BlockSpec((NC, BN, L), lambda i: (0, i, 0)),
            pl.BlockSpec((BN, 1), lambda i: (i, 0)),
            pl.BlockSpec((BN, 1), lambda i: (i, 0)),
            pl.BlockSpec((BN, 1), lambda i: (i, 0)),
            full((NG, HID)),
            full((HID, HEADS)),
            full((HID, HEADS)),
            full((2, HEADS)),
            full((HID, HID)),
            full((HID, HID)),
            full((HEADS, HID)),
            full((HID, HID)),
            full((1, HID)),
            full((1, HID)),
            full((1, HID)),
            full((HID, HID)),
            full((1, HID)),
        ],
        out_specs=pl.BlockSpec((BN, HID), lambda i: (i, 0)),
        out_shape=jax.ShapeDtypeStruct((N, HID), jnp.float32),
    )(h_E, m_part, w_part, px, py, batch2, u, Wpx, Wpy, bxy,
      W1a, W1b, W1c, W1d, b1, ln_w, ln_b, W2, b2)


def kernel(x, edge_index, edge_attr, u, batch, W_projH, b_projH, W1, b1,
           ln_w, ln_b, W2, b2):
    h_E = x[:, :HID]
    posx = x[:, HID]
    posy = x[:, HID + 1]

    V = _proj(edge_attr, W_projH, b_projH)

    rowt = edge_index[0].reshape(NB, CH)
    colt = edge_index[1].reshape(NB, CH)
    m_part = _sc_m(edge_attr, colt)
    w_part = _sc_w(V, rowt, colt, posx, posy)

    Wpx = W_projH[:, 0::2]
    Wpy = W_projH[:, 1::2]
    bxy = jnp.stack([b_projH[0::2], b_projH[1::2]])
    W1a = W1[:HID]
    W1b = W1[HID:2 * HID]
    W1c = W1[2 * HID:2 * HID + HEADS]
    W1d = W1[2 * HID + HEADS:]
    return _mlp(h_E, m_part, w_part, posx[:, None], posy[:, None],
                batch[:, None].astype(jnp.int32), u, Wpx, Wpy, bxy,
                W1a, W1b, W1c, W1d, b1.reshape(1, HID), ln_w.reshape(1, HID),
                ln_b.reshape(1, HID), W2, b2.reshape(1, HID))


# speedup vs baseline: 7.5435x; 1.3191x over previous
import functools

import jax
import jax.numpy as jnp
from jax import lax
from jax.experimental import pallas as pl
from jax.experimental.pallas import tpu as pltpu
from jax.experimental.pallas import tpu_sc as plsc

N = 10000
E = 320000
HID = 128
HEADS = 4
NG = 16

NC = 2
NS = 16
NW = NC * NS
CH = 128
NB = E // CH
NBI = -(-NB // NW)
NP = 10112
RPT = NP // NS
L = 16
NSL = 2
CHUNKS = [(k * CH, CH) for k in range(RPT // CH)]
if RPT % CH:
    CHUNKS.append((RPT - RPT % CH, RPT % CH))


def _iota16():
    return lax.broadcasted_iota(jnp.int32, (L,), 0)


def _full16(v):
    return jnp.full((L,), v, dtype=jnp.int32)


BE = 4000


def _proj_body(ea_ref, w_ref, b_ref, v_ref):
    v_ref[...] = (
        jnp.dot(ea_ref[...], w_ref[...], preferred_element_type=jnp.float32)
        + b_ref[...]
    )


def _proj(edge_attr, W_projH, b_projH):
    return pl.pallas_call(
        _proj_body,
        grid=(E // BE,),
        in_specs=[
            pl.BlockSpec((BE, HID), lambda i: (i, 0)),
            pl.BlockSpec((HID, 2 * HEADS), lambda i: (0, 0)),
            pl.BlockSpec((1, 2 * HEADS), lambda i: (0, 0)),
        ],
        out_specs=pl.BlockSpec((BE, 2 * HEADS), lambda i: (i, 0)),
        out_shape=jax.ShapeDtypeStruct((E, 2 * HEADS), jnp.float32),
    )(edge_attr, W_projH, b_projH.reshape(1, 2 * HEADS))


def _mesh():
    return plsc.VectorSubcoreMesh(
        core_axis_name="c", subcore_axis_name="s",
        num_cores=NC, num_subcores=NS)


def _sc_m_body(ea, colt, m_part, acc_m, a_buf, col_buf):
    c = lax.axis_index("c")
    s = lax.axis_index("s")
    wid = c * NS + s
    zeros = jnp.zeros((L,), jnp.float32)

    def _za(r, _):
        for k in range(HID // L):
            a_buf[r, pl.ds(k * L, L)] = zeros
        return _
    lax.fori_loop(0, CH, _za, 0)

    for off, sz in CHUNKS:
        pltpu.sync_copy(a_buf.at[pl.ds(0, sz)],
                        acc_m.at[pl.ds(s * RPT + off, sz)])

    plsc.subcore_barrier()

    def _step(i, carry):
        rw = i * NW + wid

        @pl.when(rw < NB)
        def _body():
            pltpu.sync_copy(colt.at[rw], col_buf)
            pltpu.sync_copy(ea.at[pl.ds(rw * CH, CH)], a_buf)
            pltpu.sync_copy(a_buf, acc_m.at[col_buf], add=True)
        return carry
    lax.fori_loop(0, NBI, _step, 0)

    plsc.subcore_barrier()

    for off, sz in CHUNKS:
        r0 = s * RPT + off
        pltpu.sync_copy(acc_m.at[pl.ds(r0, sz)], a_buf.at[pl.ds(0, sz)])
        pltpu.sync_copy(a_buf.at[pl.ds(0, sz)], m_part.at[c, pl.ds(r0, sz)])


def _sc_m(edge_attr, colt):
    f = pl.kernel(
        _sc_m_body,
        out_type=jax.ShapeDtypeStruct((NC, NP, HID), jnp.float32),
        mesh=_mesh(),
        scratch_types=[
            pltpu.VMEM_SHARED((NP, HID), jnp.float32),
            pltpu.VMEM((CH, HID), jnp.float32),
            pltpu.VMEM((CH,), jnp.int32),
        ],
        compiler_params=pltpu.CompilerParams(needs_layout_passes=False),
    )
    return f(edge_attr, colt)


def _sc_w_body(vv, rowt, colt, posx, posy, w_part, *refs):
    acc_w, posx_v, posy_v = refs[:3]
    v_bufs = refs[3:3 + NSL]
    pay_bufs = refs[3 + NSL:3 + 2 * NSL]
    row_bufs = refs[3 + 2 * NSL:3 + 3 * NSL]
    col_bufs = refs[3 + 3 * NSL:3 + 4 * NSL]
    f_sems = refs[3 + 4 * NSL:3 + 5 * NSL]
    c = lax.axis_index("c")
    s = lax.axis_index("s")
    wid = c * NS + s
    zeros = jnp.zeros((L,), jnp.float32)
    pay0 = pay_bufs[0]

    def _zp(r, _):
        pay0[r, :] = zeros
        return _
    lax.fori_loop(0, CH, _zp, 0)

    for off, sz in CHUNKS:
        pltpu.sync_copy(pay0.at[pl.ds(0, sz)],
                        acc_w.at[pl.ds(s * RPT + off, sz)])

    pltpu.sync_copy(posx, posx_v)
    pltpu.sync_copy(posy, posy_v)

    unit = jnp.where(_iota16() == HEADS, 1.0, 0.0).astype(jnp.float32)

    def _zu(r, _):
        for pb in pay_bufs:
            pb[r, :] = unit
        return _
    lax.fori_loop(0, CH, _zu, 0)

    plsc.subcore_barrier()

    def _fill(rw, b):
        pltpu.async_copy(rowt.at[rw], row_bufs[b], f_sems[b])
        pltpu.async_copy(colt.at[rw], col_bufs[b], f_sems[b])
        pltpu.async_copy(vv.at[pl.ds(rw * CH, CH)], v_bufs[b], f_sems[b])

    def _wait_fill(rw, b):
        pltpu.make_async_copy(rowt.at[rw], row_bufs[b], f_sems[b]).wait()
        pltpu.make_async_copy(colt.at[rw], col_bufs[b], f_sems[b]).wait()
        pltpu.make_async_copy(
            vv.at[pl.ds(rw * CH, CH)], v_bufs[b], f_sems[b]).wait()

    for b in range(NSL):
        _fill(b * NW + wid, b)

    def _group(g, carry):
        for b in range(NSL):
            rw = (g * NSL + b) * NW + wid

            @pl.when(rw < NB)
            def _body(b=b, rw=rw):
                _wait_fill(rw, b)
                for j in range(CH // L):
                    e16 = _iota16() + j * L
                    r16 = row_bufs[b][pl.ds(j * L, L)]
                    psx = plsc.load_gather(posx_v, [r16])
                    psy = plsc.load_gather(posy_v, [r16])
                    for h in range(HEADS):
                        vx = plsc.load_gather(v_bufs[b], [e16, _full16(2 * h)])
                        vy = plsc.load_gather(
                            v_bufs[b], [e16, _full16(2 * h + 1)])
                        c1 = vx * psy - vy * psx
                        plsc.store_scatter(pay_bufs[b], [e16, _full16(h)], c1)
                pltpu.sync_copy(pay_bufs[b], acc_w.at[col_bufs[b]],
                                add=True)

            rw2 = rw + NSL * NW

            @pl.when(rw2 < NB)
            def _refill(b=b, rw2=rw2):
                _fill(rw2, b)
        return carry
    lax.fori_loop(0, -(-NBI // NSL), _group, 0)

    plsc.subcore_barrier()

    for off, sz in CHUNKS:
        r0 = s * RPT + off
        pltpu.sync_copy(acc_w.at[pl.ds(r0, sz)], pay0.at[pl.ds(0, sz)])
        pltpu.sync_copy(pay0.at[pl.ds(0, sz)], w_part.at[c, pl.ds(r0, sz)])


def _sc_w(V, rowt, colt, posx, posy):
    f = pl.kernel(
        _sc_w_body,
        out_type=jax.ShapeDtypeStruct((NC, NP, L), jnp.float32),
        mesh=_mesh(),
        scratch_types=(
            [
                pltpu.VMEM_SHARED((NP, L), jnp.float32),
                pltpu.VMEM((N,), jnp.float32),
                pltpu.VMEM((N,), jnp.float32),
            ]
            + [pltpu.VMEM((CH, 2 * HEADS), jnp.float32)] * NSL
            + [pltpu.VMEM((CH, L), jnp.float32)] * NSL
            + [pltpu.VMEM((CH,), jnp.int32)] * NSL
            + [pltpu.VMEM((CH,), jnp.int32)] * NSL
            + [pltpu.SemaphoreType.DMA] * NSL
        ),
        compiler_params=pltpu.CompilerParams(needs_layout_passes=False),
    )
    return f(V, rowt, colt, posx, posy)


BN = 1000


def _mlp_body(he_ref, mp_ref, wp_ref, px_ref, py_ref, bt_ref, u_ref,
              wpx_ref, wpy_ref, bxy_ref, w1a_ref, w1b_ref, w1c_ref, w1d_ref,
              b1_ref, lnw_ref, lnb_ref, w2_ref, b2_ref, out_ref):
    m = mp_ref[0] + mp_ref[1]
    w = wp_ref[0] + wp_ref[1]
    c1 = w[:, :HEADS]
    deg = w[:, HEADS:HEADS + 1]
    bx = bxy_ref[0:1, :]
    by = bxy_ref[1:2, :]
    svx = jnp.dot(m, wpx_ref[...], preferred_element_type=jnp.float32) + deg * bx
    svy = jnp.dot(m, wpy_ref[...], preferred_element_type=jnp.float32) + deg * by
    curl = c1 + px_ref[...] * svy - py_ref[...] * svx
    onehot = (bt_ref[...] == lax.broadcasted_iota(jnp.int32, (BN, NG), 1)
              ).astype(jnp.float32)
    uw = jnp.dot(u_ref[...], w1d_ref[...], preferred_element_type=jnp.float32)
    h = (jnp.dot(he_ref[...], w1a_ref[...], preferred_element_type=jnp.float32)
         + jnp.dot(m, w1b_ref[...], preferred_element_type=jnp.float32)
         + jnp.dot(curl, w1c_ref[...], preferred_element_type=jnp.float32)
         + jnp.dot(onehot, uw, preferred_element_type=jnp.float32)
         + b1_ref[...])
    mean = jnp.mean(h, axis=-1, keepdims=True)
    var = jnp.mean((h - mean) ** 2, axis=-1, keepdims=True)
    h = (h - mean) * jax.lax.rsqrt(var + 1e-5) * lnw_ref[...] + lnb_ref[...]
    h = h * jax.nn.sigmoid(h)
    out_ref[...] = (
        jnp.dot(h, w2_ref[...], preferred_element_type=jnp.float32) + b2_ref[...])


def _mlp(h_E, m_part, w_part, px, py, batch2, u, Wpx, Wpy, bxy,
         W1a, W1b, W1c, W1d, b1, ln_w, ln_b, W2, b2):
    full = lambda shape: pl.BlockSpec(shape, lambda i: tuple(0 for _ in shape))
    return pl.pallas_call(
        _mlp_body,
        grid=(N // BN,),
        in_specs=[
            pl.BlockSpec((BN, HID), lambda i: (i, 0)),
            pl.BlockSpec((NC, BN, HID), lambda i: (0, i, 0)),
            pl.BlockSpec((NC, BN, L), lambda i: (0, i, 0)),
            pl.BlockSpec((BN, 1), lambda i: (i, 0)),
            pl.BlockSpec((BN, 1), lambda i: (i, 0)),
            pl.BlockSpec((BN, 1), lambda i: (i, 0)),
            full((NG, HID)),
            full((HID, HEADS)),
            full((HID, HEADS)),
            full((2, HEADS)),
            full((HID, HID)),
            full((HID, HID)),
            full((HEADS, HID)),
            full((HID, HID)),
            full((1, HID)),
            full((1, HID)),
            full((1, HID)),
            full((HID, HID)),
            full((1, HID)),
        ],
        out_specs=pl.BlockSpec((BN, HID), lambda i: (i, 0)),
        out_shape=jax.ShapeDtypeStruct((N, HID), jnp.float32),
    )(h_E, m_part, w_part, px, py, batch2, u, Wpx, Wpy, bxy,
      W1a, W1b, W1c, W1d, b1, ln_w, ln_b, W2, b2)


def kernel(x, edge_index, edge_attr, u, batch, W_projH, b_projH, W1, b1,
           ln_w, ln_b, W2, b2):
    h_E = x[:, :HID]
    posx = x[:, HID]
    posy = x[:, HID + 1]

    V = _proj(edge_attr, W_projH, b_projH)

    rowt = edge_index[0].reshape(NB, CH)
    colt = edge_index[1].reshape(NB, CH)
    m_part = _sc_m(edge_attr, colt)
    w_part = _sc_w(V, rowt, colt, posx, posy)

    Wpx = W_projH[:, 0::2]
    Wpy = W_projH[:, 1::2]
    bxy = jnp.stack([b_projH[0::2], b_projH[1::2]])
    W1a = W1[:HID]
    W1b = W1[HID:2 * HID]
    W1c = W1[2 * HID:2 * HID + HEADS]
    W1d = W1[2 * HID + HEADS:]
    return _mlp(h_E, m_part, w_part, posx[:, None], posy[:, None],
                batch[:, None].astype(jnp.int32), u, Wpx, Wpy, bxy,
                W1a, W1b, W1c, W1d, b1.reshape(1, HID), ln_w.reshape(1, HID),
                ln_b.reshape(1, HID), W2, b2.reshape(1, HID))


# speedup vs baseline: 9.4875x; 1.2577x over previous
import functools

import jax
import jax.numpy as jnp
from jax import lax
from jax.experimental import pallas as pl
from jax.experimental.pallas import tpu as pltpu
from jax.experimental.pallas import tpu_sc as plsc

N = 10000
E = 320000
HID = 128
HEADS = 4
NG = 16

NC = 2
NS = 16
NW = NC * NS
CH = 128
NB = E // CH
NBI = -(-NB // NW)
NP = 10112
RPT = NP // NS
L = 16
NSL = 2
CHUNKS = [(k * CH, CH) for k in range(RPT // CH)]
if RPT % CH:
    CHUNKS.append((RPT - RPT % CH, RPT % CH))


def _iota16():
    return lax.broadcasted_iota(jnp.int32, (L,), 0)


def _full16(v):
    return jnp.full((L,), v, dtype=jnp.int32)


BE = 4000


def _proj_body(ea_ref, w_ref, b_ref, v_ref):
    v_ref[...] = (
        jnp.dot(ea_ref[...], w_ref[...], preferred_element_type=jnp.float32)
        + b_ref[...]
    )


def _proj(edge_attr, W_projH, b_projH):
    return pl.pallas_call(
        _proj_body,
        grid=(E // BE,),
        in_specs=[
            pl.BlockSpec((BE, HID), lambda i: (i, 0)),
            pl.BlockSpec((HID, 2 * HEADS), lambda i: (0, 0)),
            pl.BlockSpec((1, 2 * HEADS), lambda i: (0, 0)),
        ],
        out_specs=pl.BlockSpec((BE, 2 * HEADS), lambda i: (i, 0)),
        out_shape=jax.ShapeDtypeStruct((E, 2 * HEADS), jnp.float32),
    )(edge_attr, W_projH, b_projH.reshape(1, 2 * HEADS))


def _mesh():
    return plsc.VectorSubcoreMesh(
        core_axis_name="c", subcore_axis_name="s",
        num_cores=NC, num_subcores=NS)


def _sc_m_body(ea, colt, m_part, acc_m, a_buf0, a_buf1, col_buf0, col_buf1,
               sem0, sem1):
    a_bufs = (a_buf0, a_buf1)
    col_bufs = (col_buf0, col_buf1)
    sems = (sem0, sem1)
    c = lax.axis_index("c")
    s = lax.axis_index("s")
    wid = c * NS + s
    zeros = jnp.zeros((L,), jnp.float32)

    def _za(r, _):
        for k in range(HID // L):
            a_buf0[r, pl.ds(k * L, L)] = zeros
        return _
    lax.fori_loop(0, CH, _za, 0)

    for off, sz in CHUNKS:
        pltpu.sync_copy(a_buf0.at[pl.ds(0, sz)],
                        acc_m.at[pl.ds(s * RPT + off, sz)])

    plsc.subcore_barrier()

    def _fill(rw, b):
        pltpu.async_copy(colt.at[rw], col_bufs[b], sems[b])
        pltpu.async_copy(ea.at[pl.ds(rw * CH, CH)], a_bufs[b], sems[b])

    def _wait_fill(rw, b):
        pltpu.make_async_copy(colt.at[rw], col_bufs[b], sems[b]).wait()
        pltpu.make_async_copy(
            ea.at[pl.ds(rw * CH, CH)], a_bufs[b], sems[b]).wait()

    for b in range(2):
        _fill(b * NW + wid, b)

    def _group(g, carry):
        for b in range(2):
            rw = (g * 2 + b) * NW + wid

            @pl.when(rw < NB)
            def _body(b=b, rw=rw):
                _wait_fill(rw, b)
                pltpu.sync_copy(a_bufs[b], acc_m.at[col_bufs[b]], add=True)

            rw2 = rw + 2 * NW

            @pl.when(rw2 < NB)
            def _refill(b=b, rw2=rw2):
                _fill(rw2, b)
        return carry
    lax.fori_loop(0, -(-NBI // 2), _group, 0)

    plsc.subcore_barrier()

    for off, sz in CHUNKS:
        r0 = s * RPT + off
        pltpu.sync_copy(acc_m.at[pl.ds(r0, sz)], a_buf0.at[pl.ds(0, sz)])
        pltpu.sync_copy(a_buf0.at[pl.ds(0, sz)], m_part.at[c, pl.ds(r0, sz)])


def _sc_m(edge_attr, colt):
    f = pl.kernel(
        _sc_m_body,
        out_type=jax.ShapeDtypeStruct((NC, NP, HID), jnp.float32),
        mesh=_mesh(),
        scratch_types=[
            pltpu.VMEM_SHARED((NP, HID), jnp.float32),
            pltpu.VMEM((CH, HID), jnp.float32),
            pltpu.VMEM((CH, HID), jnp.float32),
            pltpu.VMEM((CH,), jnp.int32),
            pltpu.VMEM((CH,), jnp.int32),
            pltpu.SemaphoreType.DMA,
            pltpu.SemaphoreType.DMA,
        ],
        compiler_params=pltpu.CompilerParams(needs_layout_passes=False),
    )
    return f(edge_attr, colt)


def _sc_w_body(vv, rowt, colt, posx, posy, w_part, *refs):
    acc_w, posx_v, posy_v = refs[:3]
    v_bufs = refs[3:3 + NSL]
    pay_bufs = refs[3 + NSL:3 + 2 * NSL]
    row_bufs = refs[3 + 2 * NSL:3 + 3 * NSL]
    col_bufs = refs[3 + 3 * NSL:3 + 4 * NSL]
    f_sems = refs[3 + 4 * NSL:3 + 5 * NSL]
    c = lax.axis_index("c")
    s = lax.axis_index("s")
    wid = c * NS + s
    zeros = jnp.zeros((L,), jnp.float32)
    pay0 = pay_bufs[0]

    def _zp(r, _):
        pay0[r, :] = zeros
        return _
    lax.fori_loop(0, CH, _zp, 0)

    for off, sz in CHUNKS:
        pltpu.sync_copy(pay0.at[pl.ds(0, sz)],
                        acc_w.at[pl.ds(s * RPT + off, sz)])

    pltpu.sync_copy(posx, posx_v)
    pltpu.sync_copy(posy, posy_v)

    unit = jnp.where(_iota16() == HEADS, 1.0, 0.0).astype(jnp.float32)

    def _zu(r, _):
        for pb in pay_bufs:
            pb[r, :] = unit
        return _
    lax.fori_loop(0, CH, _zu, 0)

    plsc.subcore_barrier()

    def _fill(rw, b):
        pltpu.async_copy(rowt.at[rw], row_bufs[b], f_sems[b])
        pltpu.async_copy(colt.at[rw], col_bufs[b], f_sems[b])
        pltpu.async_copy(vv.at[pl.ds(rw * CH, CH)], v_bufs[b], f_sems[b])

    def _wait_fill(rw, b):
        pltpu.make_async_copy(rowt.at[rw], row_bufs[b], f_sems[b]).wait()
        pltpu.make_async_copy(colt.at[rw], col_bufs[b], f_sems[b]).wait()
        pltpu.make_async_copy(
            vv.at[pl.ds(rw * CH, CH)], v_bufs[b], f_sems[b]).wait()

    for b in range(NSL):
        _fill(b * NW + wid, b)

    def _group(g, carry):
        for b in range(NSL):
            rw = (g * NSL + b) * NW + wid

            @pl.when(rw < NB)
            def _body(b=b, rw=rw):
                _wait_fill(rw, b)
                for j in range(CH // L):
                    e16 = _iota16() + j * L
                    r16 = row_bufs[b][pl.ds(j * L, L)]
                    psx = plsc.load_gather(posx_v, [r16])
                    psy = plsc.load_gather(posy_v, [r16])
                    for h in range(HEADS):
                        vx = plsc.load_gather(v_bufs[b], [e16, _full16(2 * h)])
                        vy = plsc.load_gather(
                            v_bufs[b], [e16, _full16(2 * h + 1)])
                        c1 = vx * psy - vy * psx
                        plsc.store_scatter(pay_bufs[b], [e16, _full16(h)], c1)
                pltpu.sync_copy(pay_bufs[b], acc_w.at[col_bufs[b]],
                                add=True)

            rw2 = rw + NSL * NW

            @pl.when(rw2 < NB)
            def _refill(b=b, rw2=rw2):
                _fill(rw2, b)
        return carry
    lax.fori_loop(0, -(-NBI // NSL), _group, 0)

    plsc.subcore_barrier()

    for off, sz in CHUNKS:
        r0 = s * RPT + off
        pltpu.sync_copy(acc_w.at[pl.ds(r0, sz)], pay0.at[pl.ds(0, sz)])
        pltpu.sync_copy(pay0.at[pl.ds(0, sz)], w_part.at[c, pl.ds(r0, sz)])


def _sc_w(V, rowt, colt, posx, posy):
    f = pl.kernel(
        _sc_w_body,
        out_type=jax.ShapeDtypeStruct((NC, NP, L), jnp.float32),
        mesh=_mesh(),
        scratch_types=(
            [
                pltpu.VMEM_SHARED((NP, L), jnp.float32),
                pltpu.VMEM((N,), jnp.float32),
                pltpu.VMEM((N,), jnp.float32),
            ]
            + [pltpu.VMEM((CH, 2 * HEADS), jnp.float32)] * NSL
            + [pltpu.VMEM((CH, L), jnp.float32)] * NSL
            + [pltpu.VMEM((CH,), jnp.int32)] * NSL
            + [pltpu.VMEM((CH,), jnp.int32)] * NSL
            + [pltpu.SemaphoreType.DMA] * NSL
        ),
        compiler_params=pltpu.CompilerParams(needs_layout_passes=False),
    )
    return f(V, rowt, colt, posx, posy)


BN = 1000


def _mlp_body(he_ref, mp_ref, wp_ref, px_ref, py_ref, bt_ref, u_ref,
              wpx_ref, wpy_ref, bxy_ref, w1a_ref, w1b_ref, w1c_ref, w1d_ref,
              b1_ref, lnw_ref, lnb_ref, w2_ref, b2_ref, out_ref):
    m = mp_ref[0] + mp_ref[1]
    w = wp_ref[0] + wp_ref[1]
    c1 = w[:, :HEADS]
    deg = w[:, HEADS:HEADS + 1]
    bx = bxy_ref[0:1, :]
    by = bxy_ref[1:2, :]
    svx = jnp.dot(m, wpx_ref[...], preferred_element_type=jnp.float32) + deg * bx
    svy = jnp.dot(m, wpy_ref[...], preferred_element_type=jnp.float32) + deg * by
    curl = c1 + px_ref[...] * svy - py_ref[...] * svx
    onehot = (bt_ref[...] == lax.broadcasted_iota(jnp.int32, (BN, NG), 1)
              ).astype(jnp.float32)
    uw = jnp.dot(u_ref[...], w1d_ref[...], preferred_element_type=jnp.float32)
    h = (jnp.dot(he_ref[...], w1a_ref[...], preferred_element_type=jnp.float32)
         + jnp.dot(m, w1b_ref[...], preferred_element_type=jnp.float32)
         + jnp.dot(curl, w1c_ref[...], preferred_element_type=jnp.float32)
         + jnp.dot(onehot, uw, preferred_element_type=jnp.float32)
         + b1_ref[...])
    mean = jnp.mean(h, axis=-1, keepdims=True)
    var = jnp.mean((h - mean) ** 2, axis=-1, keepdims=True)
    h = (h - mean) * jax.lax.rsqrt(var + 1e-5) * lnw_ref[...] + lnb_ref[...]
    h = h * jax.nn.sigmoid(h)
    out_ref[...] = (
        jnp.dot(h, w2_ref[...], preferred_element_type=jnp.float32) + b2_ref[...])


def _mlp(h_E, m_part, w_part, px, py, batch2, u, Wpx, Wpy, bxy,
         W1a, W1b, W1c, W1d, b1, ln_w, ln_b, W2, b2):
    full = lambda shape: pl.BlockSpec(shape, lambda i: tuple(0 for _ in shape))
    return pl.pallas_call(
        _mlp_body,
        grid=(N // BN,),
        in_specs=[
            pl.BlockSpec((BN, HID), lambda i: (i, 0)),
            pl.BlockSpec((NC, BN, HID), lambda i: (0, i, 0)),
            pl.BlockSpec((NC, BN, L), lambda i: (0, i, 0)),
            pl.BlockSpec((BN, 1), lambda i: (i, 0)),
            pl.BlockSpec((BN, 1), lambda i: (i, 0)),
            pl.BlockSpec((BN, 1), lambda i: (i, 0)),
            full((NG, HID)),
            full((HID, HEADS)),
            full((HID, HEADS)),
            full((2, HEADS)),
            full((HID, HID)),
            full((HID, HID)),
            full((HEADS, HID)),
            full((HID, HID)),
            full((1, HID)),
            full((1, HID)),
            full((1, HID)),
            full((HID, HID)),
            full((1, HID)),
        ],
        out_specs=pl.BlockSpec((BN, HID), lambda i: (i, 0)),
        out_shape=jax.ShapeDtypeStruct((N, HID), jnp.float32),
    )(h_E, m_part, w_part, px, py, batch2, u, Wpx, Wpy, bxy,
      W1a, W1b, W1c, W1d, b1, ln_w, ln_b, W2, b2)


def kernel(x, edge_index, edge_attr, u, batch, W_projH, b_projH, W1, b1,
           ln_w, ln_b, W2, b2):
    h_E = x[:, :HID]
    posx = x[:, HID]
    posy = x[:, HID + 1]

    V = _proj(edge_attr, W_projH, b_projH)

    rowt = edge_index[0].reshape(NB, CH)
    colt = edge_index[1].reshape(NB, CH)
    m_part = _sc_m(edge_attr, colt)
    w_part = _sc_w(V, rowt, colt, posx, posy)

    Wpx = W_projH[:, 0::2]
    Wpy = W_projH[:, 1::2]
    bxy = jnp.stack([b_projH[0::2], b_projH[1::2]])
    W1a = W1[:HID]
    W1b = W1[HID:2 * HID]
    W1c = W1[2 * HID:2 * HID + HEADS]
    W1d = W1[2 * HID + HEADS:]
    return _mlp(h_E, m_part, w_part, posx[:, None], posy[:, None],
                batch[:, None].astype(jnp.int32), u, Wpx, Wpy, bxy,
                W1a, W1b, W1c, W1d, b1.reshape(1, HID), ln_w.reshape(1, HID),
                ln_b.reshape(1, HID), W2, b2.reshape(1, HID))


# speedup vs baseline: 9.4976x; 1.0011x over previous
import functools

import jax
import jax.numpy as jnp
from jax import lax
from jax.experimental import pallas as pl
from jax.experimental.pallas import tpu as pltpu
from jax.experimental.pallas import tpu_sc as plsc

N = 10000
E = 320000
HID = 128
HEADS = 4
NG = 16

NC = 2
NS = 16
NW = NC * NS
CH = 128
NB = E // CH
NBI = -(-NB // NW)
NP = 10112
RPT = NP // NS
L = 16
NSL = 2
CHUNKS = [(k * CH, CH) for k in range(RPT // CH)]
if RPT % CH:
    CHUNKS.append((RPT - RPT % CH, RPT % CH))


def _iota16():
    return lax.broadcasted_iota(jnp.int32, (L,), 0)


def _full16(v):
    return jnp.full((L,), v, dtype=jnp.int32)


BE = 4000


def _proj_body(ea_ref, w_ref, b_ref, v_ref):
    v_ref[...] = (
        jnp.dot(ea_ref[...], w_ref[...], preferred_element_type=jnp.float32)
        + b_ref[...]
    )


def _proj(edge_attr, W_projH, b_projH):
    return pl.pallas_call(
        _proj_body,
        grid=(E // BE,),
        in_specs=[
            pl.BlockSpec((BE, HID), lambda i: (i, 0)),
            pl.BlockSpec((HID, 2 * HEADS), lambda i: (0, 0)),
            pl.BlockSpec((1, 2 * HEADS), lambda i: (0, 0)),
        ],
        out_specs=pl.BlockSpec((BE, 2 * HEADS), lambda i: (i, 0)),
        out_shape=jax.ShapeDtypeStruct((E, 2 * HEADS), jnp.float32),
    )(edge_attr, W_projH, b_projH.reshape(1, 2 * HEADS))


def _mesh():
    return plsc.VectorSubcoreMesh(
        core_axis_name="c", subcore_axis_name="s",
        num_cores=NC, num_subcores=NS)


def _sc_m_body(ea, colt, m_part, acc_m, a_buf0, a_buf1, col_buf0, col_buf1,
               sem0, sem1):
    a_bufs = (a_buf0, a_buf1)
    col_bufs = (col_buf0, col_buf1)
    sems = (sem0, sem1)
    c = lax.axis_index("c")
    s = lax.axis_index("s")
    wid = c * NS + s
    zeros = jnp.zeros((L,), jnp.float32)

    def _za(r, _):
        for k in range(HID // L):
            a_buf0[r, pl.ds(k * L, L)] = zeros
        return _
    lax.fori_loop(0, CH, _za, 0)

    for off, sz in CHUNKS:
        pltpu.sync_copy(a_buf0.at[pl.ds(0, sz)],
                        acc_m.at[pl.ds(s * RPT + off, sz)])

    plsc.subcore_barrier()

    def _fill(rw, b):
        pltpu.async_copy(colt.at[rw], col_bufs[b], sems[b])
        pltpu.async_copy(ea.at[pl.ds(rw * CH, CH)], a_bufs[b], sems[b])

    def _wait_fill(rw, b):
        pltpu.make_async_copy(colt.at[rw], col_bufs[b], sems[b]).wait()
        pltpu.make_async_copy(
            ea.at[pl.ds(rw * CH, CH)], a_bufs[b], sems[b]).wait()

    for b in range(2):
        _fill(b * NW + wid, b)

    def _group(g, carry):
        for b in range(2):
            rw = (g * 2 + b) * NW + wid

            @pl.when(rw < NB)
            def _body(b=b, rw=rw):
                _wait_fill(rw, b)
                pltpu.sync_copy(a_bufs[b], acc_m.at[col_bufs[b]], add=True)

            rw2 = rw + 2 * NW

            @pl.when(rw2 < NB)
            def _refill(b=b, rw2=rw2):
                _fill(rw2, b)
        return carry
    lax.fori_loop(0, -(-NBI // 2), _group, 0)

    plsc.subcore_barrier()

    for off, sz in CHUNKS:
        r0 = s * RPT + off
        pltpu.sync_copy(acc_m.at[pl.ds(r0, sz)], a_buf0.at[pl.ds(0, sz)])
        pltpu.sync_copy(a_buf0.at[pl.ds(0, sz)], m_part.at[c, pl.ds(r0, sz)])


def _sc_m(edge_attr, colt):
    f = pl.kernel(
        _sc_m_body,
        out_type=jax.ShapeDtypeStruct((NC, NP, HID), jnp.float32),
        mesh=_mesh(),
        scratch_types=[
            pltpu.VMEM_SHARED((NP, HID), jnp.float32),
            pltpu.VMEM((CH, HID), jnp.float32),
            pltpu.VMEM((CH, HID), jnp.float32),
            pltpu.VMEM((CH,), jnp.int32),
            pltpu.VMEM((CH,), jnp.int32),
            pltpu.SemaphoreType.DMA,
            pltpu.SemaphoreType.DMA,
        ],
        compiler_params=pltpu.CompilerParams(needs_layout_passes=False),
    )
    return f(edge_attr, colt)


def _sc_w_body(vv, rowt, colt, posx, posy, w_part, *refs):
    acc_w, posx_v, posy_v = refs[:3]
    v_bufs = refs[3:3 + NSL]
    pay_bufs = refs[3 + NSL:3 + 2 * NSL]
    row_bufs = refs[3 + 2 * NSL:3 + 3 * NSL]
    col_bufs = refs[3 + 3 * NSL:3 + 4 * NSL]
    f_sems = refs[3 + 4 * NSL:3 + 5 * NSL]
    c = lax.axis_index("c")
    s = lax.axis_index("s")
    wid = c * NS + s
    zeros = jnp.zeros((L,), jnp.float32)
    pay0 = pay_bufs[0]

    def _zp(r, _):
        pay0[r, :] = zeros
        return _
    lax.fori_loop(0, CH, _zp, 0)

    for off, sz in CHUNKS:
        pltpu.sync_copy(pay0.at[pl.ds(0, sz)],
                        acc_w.at[pl.ds(s * RPT + off, sz)])

    pltpu.sync_copy(posx, posx_v)
    pltpu.sync_copy(posy, posy_v)

    unit = jnp.where(_iota16() == HEADS, 1.0, 0.0).astype(jnp.float32)

    def _zu(r, _):
        for pb in pay_bufs:
            pb[r, :] = unit
        return _
    lax.fori_loop(0, CH, _zu, 0)

    plsc.subcore_barrier()

    def _fill(rw, b):
        pltpu.async_copy(rowt.at[rw], row_bufs[b], f_sems[b])
        pltpu.async_copy(colt.at[rw], col_bufs[b], f_sems[b])
        pltpu.async_copy(vv.at[pl.ds(rw * CH, CH)], v_bufs[b], f_sems[b])

    def _wait_fill(rw, b):
        pltpu.make_async_copy(rowt.at[rw], row_bufs[b], f_sems[b]).wait()
        pltpu.make_async_copy(colt.at[rw], col_bufs[b], f_sems[b]).wait()
        pltpu.make_async_copy(
            vv.at[pl.ds(rw * CH, CH)], v_bufs[b], f_sems[b]).wait()

    for b in range(NSL):
        _fill(b * NW + wid, b)

    def _group(g, carry):
        for b in range(NSL):
            rw = (g * NSL + b) * NW + wid

            @pl.when(rw < NB)
            def _body(b=b, rw=rw):
                _wait_fill(rw, b)
                for j in range(CH // L):
                    e16 = _iota16() + j * L
                    r16 = row_bufs[b][pl.ds(j * L, L)]
                    psx = plsc.load_gather(posx_v, [r16])
                    psy = plsc.load_gather(posy_v, [r16])
                    for h in range(HEADS):
                        vx = plsc.load_gather(v_bufs[b], [e16, _full16(2 * h)])
                        vy = plsc.load_gather(
                            v_bufs[b], [e16, _full16(2 * h + 1)])
                        c1 = vx * psy - vy * psx
                        plsc.store_scatter(pay_bufs[b], [e16, _full16(h)], c1)
                pltpu.sync_copy(pay_bufs[b], acc_w.at[col_bufs[b]],
                                add=True)

            rw2 = rw + NSL * NW

            @pl.when(rw2 < NB)
            def _refill(b=b, rw2=rw2):
                _fill(rw2, b)
        return carry
    lax.fori_loop(0, -(-NBI // NSL), _group, 0)

    plsc.subcore_barrier()

    for off, sz in CHUNKS:
        r0 = s * RPT + off
        pltpu.sync_copy(acc_w.at[pl.ds(r0, sz)], pay0.at[pl.ds(0, sz)])
        pltpu.sync_copy(pay0.at[pl.ds(0, sz)], w_part.at[c, pl.ds(r0, sz)])


def _sc_w(V, rowt, colt, posx, posy):
    f = pl.kernel(
        _sc_w_body,
        out_type=jax.ShapeDtypeStruct((NC, NP, L), jnp.float32),
        mesh=_mesh(),
        scratch_types=(
            [
                pltpu.VMEM_SHARED((NP, L), jnp.float32),
                pltpu.VMEM((N,), jnp.float32),
                pltpu.VMEM((N,), jnp.float32),
            ]
            + [pltpu.VMEM((CH, 2 * HEADS), jnp.float32)] * NSL
            + [pltpu.VMEM((CH, L), jnp.float32)] * NSL
            + [pltpu.VMEM((CH,), jnp.int32)] * NSL
            + [pltpu.VMEM((CH,), jnp.int32)] * NSL
            + [pltpu.SemaphoreType.DMA] * NSL
        ),
        compiler_params=pltpu.CompilerParams(needs_layout_passes=False),
    )
    return f(V, rowt, colt, posx, posy)


BN = 1000


def _mlp_body(he_ref, mp_ref, wp_ref, px_ref, py_ref, bt_ref, u_ref,
              wpx_ref, wpy_ref, bxy_ref, w1a_ref, w1b_ref, w1c_ref, w1d_ref,
              b1_ref, lnw_ref, lnb_ref, w2_ref, b2_ref, out_ref):
    m = mp_ref[0] + mp_ref[1]
    w = wp_ref[0] + wp_ref[1]
    c1 = w[:, :HEADS]
    deg = w[:, HEADS:HEADS + 1]
    bx = bxy_ref[0:1, :]
    by = bxy_ref[1:2, :]
    svx = jnp.dot(m, wpx_ref[...], preferred_element_type=jnp.float32) + deg * bx
    svy = jnp.dot(m, wpy_ref[...], preferred_element_type=jnp.float32) + deg * by
    curl = c1 + px_ref[...] * svy - py_ref[...] * svx
    onehot = (bt_ref[...] == lax.broadcasted_iota(jnp.int32, (BN, NG), 1)
              ).astype(jnp.float32)
    uw = jnp.dot(u_ref[...], w1d_ref[...], preferred_element_type=jnp.float32)
    h = (jnp.dot(he_ref[...], w1a_ref[...], preferred_element_type=jnp.float32)
         + jnp.dot(m, w1b_ref[...], preferred_element_type=jnp.float32)
         + jnp.dot(curl, w1c_ref[...], preferred_element_type=jnp.float32)
         + jnp.dot(onehot, uw, preferred_element_type=jnp.float32)
         + b1_ref[...])
    mean = jnp.mean(h, axis=-1, keepdims=True)
    var = jnp.mean((h - mean) ** 2, axis=-1, keepdims=True)
    h = (h - mean) * jax.lax.rsqrt(var + 1e-5) * lnw_ref[...] + lnb_ref[...]
    h = h * jax.nn.sigmoid(h)
    out_ref[...] = (
        jnp.dot(h, w2_ref[...], preferred_element_type=jnp.float32) + b2_ref[...])


def _mlp(h_E, m_part, w_part, px, py, batch2, u, Wpx, Wpy, bxy,
         W1a, W1b, W1c, W1d, b1, ln_w, ln_b, W2, b2):
    full = lambda shape: pl.BlockSpec(shape, lambda i: tuple(0 for _ in shape))
    return pl.pallas_call(
        _mlp_body,
        grid=(N // BN,),
        in_specs=[
            pl.BlockSpec((BN, HID), lambda i: (i, 0)),
            pl.BlockSpec((NC, BN, HID), lambda i: (0, i, 0)),
            pl.BlockSpec((NC, BN, L), lambda i: (0, i, 0)),
            pl.BlockSpec((BN, 1), lambda i: (i, 0)),
            pl.BlockSpec((BN, 1), lambda i: (i, 0)),
            pl.BlockSpec((BN, 1), lambda i: (i, 0)),
            full((NG, HID)),
            full((HID, HEADS)),
            full((HID, HEADS)),
            full((2, HEADS)),
            full((HID, HID)),
            full((HID, HID)),
            full((HEADS, HID)),
            full((HID, HID)),
            full((1, HID)),
            full((1, HID)),
            full((1, HID)),
            full((HID, HID)),
            full((1, HID)),
        ],
        out_specs=pl.BlockSpec((BN, HID), lambda i: (i, 0)),
        out_shape=jax.ShapeDtypeStruct((N, HID), jnp.float32),
    )(h_E, m_part, w_part, px, py, batch2, u, Wpx, Wpy, bxy,
      W1a, W1b, W1c, W1d, b1, ln_w, ln_b, W2, b2)


def kernel(x, edge_index, edge_attr, u, batch, W_projH, b_projH, W1, b1,
           ln_w, ln_b, W2, b2):
    h_E = x[:, :HID]
    posx = x[:, HID]
    posy = x[:, HID + 1]

    rowt = edge_index[0].reshape(NB, CH)
    colt = edge_index[1].reshape(NB, CH)
    m_part = _sc_m(edge_attr, colt)
    V = _proj(edge_attr, W_projH, b_projH)
    w_part = _sc_w(V, rowt, colt, posx, posy)

    Wpx = W_projH[:, 0::2]
    Wpy = W_projH[:, 1::2]
    bxy = jnp.stack([b_projH[0::2], b_projH[1::2]])
    W1a = W1[:HID]
    W1b = W1[HID:2 * HID]
    W1c = W1[2 * HID:2 * HID + HEADS]
    W1d = W1[2 * HID + HEADS:]
    return _mlp(h_E, m_part, w_part, posx[:, None], posy[:, None],
                batch[:, None].astype(jnp.int32), u, Wpx, Wpy, bxy,
                W1a, W1b, W1c, W1d, b1.reshape(1, HID), ln_w.reshape(1, HID),
                ln_b.reshape(1, HID), W2, b2.reshape(1, HID))
